# baseline (speedup 1.0000x reference)
import jax
import jax.numpy as jnp
from jax.experimental import pallas as pl
from jax.experimental.pallas import tpu as pltpu

N = 131072
D = 64
L = 64
SEG = 1024
NSEG = 128
PCH = 8


def _log2(v):
    return v.bit_length() - 1


def _sort_column(zt_in, seg, nseg):
    log_seg = _log2(seg)
    log_n = _log2(seg * nseg)

    z = zt_in.T
    r_io = jax.lax.broadcasted_iota(jnp.int32, (seg, 1), 0)
    s_io = jax.lax.broadcasted_iota(jnp.int32, (1, nseg), 1)
    idx = s_io * seg + r_io

    def cmpex(z, k_log, bitj, pdn, pup):
        partner = jnp.where(bitj, pdn, pup)
        mn = jnp.minimum(z, partner)
        mx = jnp.maximum(z, partner)
        asc = (jax.lax.shift_right_logical(idx, k_log) & 1) == 0
        return jnp.where(asc != bitj, mn, mx)

    def sublane_pass(z, k_log, j_log):
        j = jax.lax.shift_left(jnp.int32(1), j_log)
        pdn = pltpu.roll(z, j, 0)
        pup = pltpu.roll(z, seg - j, 0)
        bitj = (jax.lax.shift_right_logical(r_io, j_log) & 1) == 1
        return cmpex(z, k_log, bitj, pdn, pup)

    def lane_pass(z, k_log, j_log):
        d_log = j_log - log_seg
        d = jax.lax.shift_left(jnp.int32(1), d_log)
        pdn = pltpu.roll(z, d, 1)
        pup = pltpu.roll(z, nseg - d, 1)
        bitd = (jax.lax.shift_right_logical(s_io, d_log) & 1) == 1
        return cmpex(z, k_log, bitd, pdn, pup)

    def stage(k_log, z):
        def inner(i, z):
            j_log = k_log - 1 - i
            return jax.lax.cond(
                j_log >= log_seg,
                lambda zz: lane_pass(zz, k_log, j_log),
                lambda zz: sublane_pass(zz, k_log, j_log),
                z)
        return jax.lax.fori_loop(0, k_log, inner, z)

    return jax.lax.fori_loop(1, log_n + 1, stage, z)


def _make_proj_body(seg, pch, d, l):
    def body(x_ref, y_ref, th_ref, a_ref):
        th = th_ref[...]
        norm = jnp.sqrt(jnp.sum(th * th, axis=0, keepdims=True))
        thn = th / (norm + 1e-12)

        def proj_t(v):
            p = jax.lax.dot_general(
                v, thn, (((1,), (0,)), ((), ())),
                precision=jax.lax.Precision.HIGHEST)
            return p.T.reshape(l, pch, seg)

        a_ref[0:l] = proj_t(x_ref[...])
        a_ref[l:2 * l] = proj_t(y_ref[...])

    return body


def _make_sort_body(seg, nseg, l):
    def body(ax_ref, ay_ref, out_ref):
        zx = _sort_column(ax_ref[0], seg, nseg)
        zy = _sort_column(ay_ref[0], seg, nseg)
        out_ref[...] = jnp.sum(jnp.abs(zx - zy)).reshape(1, 1, 1)

    return body


def _gswd(x, y, theta, seg, nseg, pch, d, l):
    n = seg * nseg
    a = pl.pallas_call(
        _make_proj_body(seg, pch, d, l),
        grid=(nseg // pch,),
        in_specs=[
            pl.BlockSpec((pch * seg, d), lambda i: (i, 0)),
            pl.BlockSpec((pch * seg, d), lambda i: (i, 0)),
            pl.BlockSpec((d, l), lambda i: (0, 0)),
        ],
        out_specs=pl.BlockSpec((2 * l, pch, seg), lambda i: (0, i, 0)),
        out_shape=jax.ShapeDtypeStruct((2 * l, nseg, seg), jnp.float32),
        compiler_params=pltpu.CompilerParams(
            dimension_semantics=("parallel",),
        ),
    )(x, y, theta)

    partials = pl.pallas_call(
        _make_sort_body(seg, nseg, l),
        grid=(l,),
        in_specs=[
            pl.BlockSpec((1, nseg, seg), lambda c: (c, 0, 0)),
            pl.BlockSpec((1, nseg, seg), lambda c: (c + l, 0, 0)),
        ],
        out_specs=pl.BlockSpec((1, 1, 1), lambda c: (c, 0, 0)),
        out_shape=jax.ShapeDtypeStruct((l, 1, 1), jnp.float32),
        compiler_params=pltpu.CompilerParams(
            dimension_semantics=("parallel",),
        ),
    )(a, a)
    return jnp.sum(partials) / (n * l)


def kernel(x, y, theta):
    return _gswd(x, y, theta, SEG, NSEG, PCH, D, L)


# speedup vs baseline: 1.2114x; 1.2114x over previous
import jax
import jax.numpy as jnp
from jax.experimental import pallas as pl
from jax.experimental.pallas import tpu as pltpu

N = 131072
D = 64
L = 64
SEG = 1024
NSEG = 128
PCH = 8


def _log2(v):
    return v.bit_length() - 1


def _sort_column(zt_in, seg, nseg):
    log_seg = _log2(seg)
    log_n = _log2(seg * nseg)

    z = zt_in.T
    r_io = jax.lax.broadcasted_iota(jnp.int32, (seg, 1), 0)
    s_io = jax.lax.broadcasted_iota(jnp.int32, (1, nseg), 1)
    idx = s_io * seg + r_io

    def sublane_pass(z, j_log):
        j = jax.lax.shift_left(jnp.int32(1), j_log)
        up = pltpu.roll(z, seg - j, 0)
        dn = pltpu.roll(z, j, 0)
        bitj = (jax.lax.shift_right_logical(r_io, j_log) & 1) == 1
        partner = jnp.where(bitj, dn, up)
        mn = jnp.minimum(z, partner)
        mx = jnp.maximum(z, partner)
        return jnp.where(bitj, mx, mn)

    def lane_pass(z, d_log):
        d = jax.lax.shift_left(jnp.int32(1), d_log)
        up = pltpu.roll(z, nseg - d, 1)
        dn = pltpu.roll(z, d, 1)
        bitd = (jax.lax.shift_right_logical(s_io, d_log) & 1) == 1
        partner = jnp.where(bitd, dn, up)
        mn = jnp.minimum(z, partner)
        mx = jnp.maximum(z, partner)
        return jnp.where(bitd, mx, mn)

    def stage(k_log, z):
        desc = (jax.lax.shift_right_logical(idx, k_log) & 1) == 1
        sgn = jnp.where(desc, jnp.float32(-1.0), jnp.float32(1.0))
        z = z * sgn
        n_lane = k_log - log_seg
        z = jax.lax.fori_loop(
            0, n_lane,
            lambda i, zz: lane_pass(zz, k_log - 1 - log_seg - i), z)
        n_sub = jnp.minimum(k_log, log_seg)
        z = jax.lax.fori_loop(
            0, n_sub,
            lambda i, zz: sublane_pass(zz, n_sub - 1 - i), z)
        return z * sgn

    return jax.lax.fori_loop(1, log_n + 1, stage, z)


def _make_proj_body(seg, pch, d, l):
    def body(x_ref, y_ref, th_ref, a_ref):
        th = th_ref[...]
        norm = jnp.sqrt(jnp.sum(th * th, axis=0, keepdims=True))
        thn = th / (norm + 1e-12)

        def proj_t(v):
            p = jax.lax.dot_general(
                v, thn, (((1,), (0,)), ((), ())),
                precision=jax.lax.Precision.HIGHEST)
            return p.T.reshape(l, pch, seg)

        a_ref[0:l] = proj_t(x_ref[...])
        a_ref[l:2 * l] = proj_t(y_ref[...])

    return body


def _make_sort_body(seg, nseg, l):
    def body(ax_ref, ay_ref, out_ref):
        zx = _sort_column(ax_ref[0], seg, nseg)
        zy = _sort_column(ay_ref[0], seg, nseg)
        out_ref[...] = jnp.sum(jnp.abs(zx - zy)).reshape(1, 1, 1)

    return body


def _gswd(x, y, theta, seg, nseg, pch, d, l):
    n = seg * nseg
    a = pl.pallas_call(
        _make_proj_body(seg, pch, d, l),
        grid=(nseg // pch,),
        in_specs=[
            pl.BlockSpec((pch * seg, d), lambda i: (i, 0)),
            pl.BlockSpec((pch * seg, d), lambda i: (i, 0)),
            pl.BlockSpec((d, l), lambda i: (0, 0)),
        ],
        out_specs=pl.BlockSpec((2 * l, pch, seg), lambda i: (0, i, 0)),
        out_shape=jax.ShapeDtypeStruct((2 * l, nseg, seg), jnp.float32),
        compiler_params=pltpu.CompilerParams(
            dimension_semantics=("parallel",),
        ),
    )(x, y, theta)

    partials = pl.pallas_call(
        _make_sort_body(seg, nseg, l),
        grid=(l,),
        in_specs=[
            pl.BlockSpec((1, nseg, seg), lambda c: (c, 0, 0)),
            pl.BlockSpec((1, nseg, seg), lambda c: (c + l, 0, 0)),
        ],
        out_specs=pl.BlockSpec((1, 1, 1), lambda c: (c, 0, 0)),
        out_shape=jax.ShapeDtypeStruct((l, 1, 1), jnp.float32),
        compiler_params=pltpu.CompilerParams(
            dimension_semantics=("parallel",),
        ),
    )(a, a)
    return jnp.sum(partials) / (n * l)


def kernel(x, y, theta):
    return _gswd(x, y, theta, SEG, NSEG, PCH, D, L)


# speedup vs baseline: 2.7553x; 2.2745x over previous
import jax
import jax.numpy as jnp
from jax.experimental import pallas as pl
from jax.experimental.pallas import tpu as pltpu

N = 131072
D = 64
L = 64
SEG = 1024
NSEG = 128
PCH = 8


def _log2(v):
    return v.bit_length() - 1


_CH = 128


def _static_pass(a, j, nseg):
    rows = a.shape[0]
    groups = rows // (2 * j)
    a4 = a.reshape(groups, 2, j, nseg)
    lower = a4[:, 0]
    upper = a4[:, 1]
    return jnp.stack(
        [jnp.minimum(lower, upper), jnp.maximum(lower, upper)],
        axis=1).reshape(rows, nseg)


def _sort_column(zt_in, zref, seg, nseg):
    log_seg = _log2(seg)
    log_n = _log2(seg * nseg)
    ch = min(_CH, seg)
    log_ch = _log2(ch)
    nch = seg // ch

    r_io = jax.lax.broadcasted_iota(jnp.int32, (seg, 1), 0)
    s_io = jax.lax.broadcasted_iota(jnp.int32, (1, nseg), 1)
    idx = s_io * seg + r_io

    zref[...] = zt_in.T

    rl_io = jax.lax.broadcasted_iota(jnp.int32, (ch, 1), 0)
    stage_sgn = {
        k: jnp.where((jax.lax.shift_right_logical(rl_io, k) & 1) == 1,
                     jnp.float32(-1.0), jnp.float32(1.0))
        for k in range(1, log_ch)
    }

    def chunk_sort(c, carry):
        row0 = c * ch
        a = zref[pl.ds(row0, ch), :]
        csgn = jnp.where(
            (jax.lax.shift_right_logical(s_io * seg + row0, log_ch) & 1) == 1,
            jnp.float32(-1.0), jnp.float32(1.0))
        a = a * csgn
        for k in range(1, log_ch + 1):
            if k < log_ch:
                a = a * stage_sgn[k]
            for j_log in range(k - 1, -1, -1):
                a = _static_pass(a, 1 << j_log, nseg)
            if k < log_ch:
                a = a * stage_sgn[k]
        a = a * csgn
        zref[pl.ds(row0, ch), :] = a
        return carry

    jax.lax.fori_loop(0, nch, chunk_sort, 0)

    def sublane_pass(z, j_log):
        j = jax.lax.shift_left(jnp.int32(1), j_log)
        up = pltpu.roll(z, seg - j, 0)
        dn = pltpu.roll(z, j, 0)
        bitj = (jax.lax.shift_right_logical(r_io, j_log) & 1) == 1
        partner = jnp.where(bitj, dn, up)
        mn = jnp.minimum(z, partner)
        mx = jnp.maximum(z, partner)
        return jnp.where(bitj, mx, mn)

    def lane_pass(z, d_log):
        d = jax.lax.shift_left(jnp.int32(1), d_log)
        up = pltpu.roll(z, nseg - d, 1)
        dn = pltpu.roll(z, d, 1)
        bitd = (jax.lax.shift_right_logical(s_io, d_log) & 1) == 1
        partner = jnp.where(bitd, dn, up)
        mn = jnp.minimum(z, partner)
        mx = jnp.maximum(z, partner)
        return jnp.where(bitd, mx, mn)

    def chunk_tail(c, carry):
        row0 = c * ch
        a = zref[pl.ds(row0, ch), :]
        for j_log in range(log_ch - 1, -1, -1):
            a = _static_pass(a, 1 << j_log, nseg)
        zref[pl.ds(row0, ch), :] = a
        return carry

    def stage(k_log, carry):
        desc = (jax.lax.shift_right_logical(idx, k_log) & 1) == 1
        sgn = jnp.where(desc, jnp.float32(-1.0), jnp.float32(1.0))
        zref[...] = zref[...] * sgn
        n_lane = k_log - log_seg
        def lane_body(i, cc):
            zref[...] = lane_pass(zref[...], k_log - 1 - log_seg - i)
            return cc
        jax.lax.fori_loop(0, n_lane, lane_body, 0)
        n_big = jnp.maximum(jnp.minimum(k_log, log_seg) - log_ch, 0)
        def big_body(i, cc):
            zref[...] = sublane_pass(
                zref[...], jnp.minimum(k_log, log_seg) - 1 - i)
            return cc
        jax.lax.fori_loop(0, n_big, big_body, 0)
        jax.lax.fori_loop(0, nch, chunk_tail, 0)
        zref[...] = zref[...] * sgn
        return carry

    jax.lax.fori_loop(log_ch + 1, log_n + 1, stage, 0)


def _make_proj_body(seg, pch, d, l):
    def body(x_ref, y_ref, th_ref, a_ref):
        th = th_ref[...]
        norm = jnp.sqrt(jnp.sum(th * th, axis=0, keepdims=True))
        thn = th / (norm + 1e-12)

        def proj_t(v):
            p = jax.lax.dot_general(
                v, thn, (((1,), (0,)), ((), ())),
                precision=jax.lax.Precision.HIGHEST)
            return p.T.reshape(l, pch, seg)

        a_ref[0:l] = proj_t(x_ref[...])
        a_ref[l:2 * l] = proj_t(y_ref[...])

    return body


def _make_sort_body(seg, nseg, l):
    def body(ax_ref, ay_ref, out_ref, zref, xres):
        _sort_column(ax_ref[0], zref, seg, nseg)
        xres[...] = zref[...]
        _sort_column(ay_ref[0], zref, seg, nseg)
        out_ref[...] = jnp.sum(
            jnp.abs(xres[...] - zref[...])).reshape(1, 1, 1)

    return body


def _gswd(x, y, theta, seg, nseg, pch, d, l):
    n = seg * nseg
    a = pl.pallas_call(
        _make_proj_body(seg, pch, d, l),
        grid=(nseg // pch,),
        in_specs=[
            pl.BlockSpec((pch * seg, d), lambda i: (i, 0)),
            pl.BlockSpec((pch * seg, d), lambda i: (i, 0)),
            pl.BlockSpec((d, l), lambda i: (0, 0)),
        ],
        out_specs=pl.BlockSpec((2 * l, pch, seg), lambda i: (0, i, 0)),
        out_shape=jax.ShapeDtypeStruct((2 * l, nseg, seg), jnp.float32),
        compiler_params=pltpu.CompilerParams(
            dimension_semantics=("parallel",),
        ),
    )(x, y, theta)

    partials = pl.pallas_call(
        _make_sort_body(seg, nseg, l),
        grid=(l,),
        in_specs=[
            pl.BlockSpec((1, nseg, seg), lambda c: (c, 0, 0)),
            pl.BlockSpec((1, nseg, seg), lambda c: (c + l, 0, 0)),
        ],
        out_specs=pl.BlockSpec((1, 1, 1), lambda c: (c, 0, 0)),
        out_shape=jax.ShapeDtypeStruct((l, 1, 1), jnp.float32),
        scratch_shapes=[
            pltpu.VMEM((seg, nseg), jnp.float32),
            pltpu.VMEM((seg, nseg), jnp.float32),
        ],
        compiler_params=pltpu.CompilerParams(
            dimension_semantics=("parallel",),
        ),
    )(a, a)
    return jnp.sum(partials) / (n * l)


def kernel(x, y, theta):
    return _gswd(x, y, theta, SEG, NSEG, PCH, D, L)


# speedup vs baseline: 2.9140x; 1.0576x over previous
import jax
import jax.numpy as jnp
from jax.experimental import pallas as pl
from jax.experimental.pallas import tpu as pltpu

N = 131072
D = 64
L = 64
SEG = 1024
NSEG = 128
PCH = 8


def _log2(v):
    return v.bit_length() - 1


_CH = 128


def _static_pass(a, j, nseg):
    rows = a.shape[0]
    groups = rows // (2 * j)
    a4 = a.reshape(groups, 2, j, nseg)
    lower = a4[:, 0]
    upper = a4[:, 1]
    return jnp.stack(
        [jnp.minimum(lower, upper), jnp.maximum(lower, upper)],
        axis=1).reshape(rows, nseg)


def _sort_column(zt_in, zref, seg, nseg):
    log_seg = _log2(seg)
    log_n = _log2(seg * nseg)
    ch = min(_CH, seg)
    log_ch = _log2(ch)
    nch = seg // ch

    r_io = jax.lax.broadcasted_iota(jnp.int32, (seg, 1), 0)
    s_io = jax.lax.broadcasted_iota(jnp.int32, (1, nseg), 1)
    idx = s_io * seg + r_io

    zref[...] = zt_in.T

    rl_io = jax.lax.broadcasted_iota(jnp.int32, (ch, 1), 0)
    stage_sgn = {
        k: jnp.where((jax.lax.shift_right_logical(rl_io, k) & 1) == 1,
                     jnp.float32(-1.0), jnp.float32(1.0))
        for k in range(1, log_ch)
    }

    def chunk_sort(c, carry):
        row0 = c * ch
        a = zref[pl.ds(row0, ch), :]
        csgn = jnp.where(
            (jax.lax.shift_right_logical(s_io * seg + row0, log_ch) & 1) == 1,
            jnp.float32(-1.0), jnp.float32(1.0))
        a = a * csgn
        for k in range(1, log_ch + 1):
            if k < log_ch:
                a = a * stage_sgn[k]
            for j_log in range(k - 1, -1, -1):
                a = _static_pass(a, 1 << j_log, nseg)
            if k < log_ch:
                a = a * stage_sgn[k]
        a = a * csgn
        zref[pl.ds(row0, ch), :] = a
        return carry

    jax.lax.fori_loop(0, nch, chunk_sort, 0)

    def lane_pass_chunk(c, d_log):
        row0 = c * ch
        a = zref[pl.ds(row0, ch), :]
        d = jax.lax.shift_left(jnp.int32(1), d_log)
        up = pltpu.roll(a, nseg - d, 1)
        dn = pltpu.roll(a, d, 1)
        bitd = (jax.lax.shift_right_logical(s_io, d_log) & 1) == 1
        partner = jnp.where(bitd, dn, up)
        mn = jnp.minimum(a, partner)
        mx = jnp.maximum(a, partner)
        zref[pl.ds(row0, ch), :] = jnp.where(bitd, mx, mn)

    def big_pass_pair(t, j_log):
        dl = j_log - log_ch
        g = jax.lax.shift_right_logical(t, dl)
        o = t & (jax.lax.shift_left(jnp.int32(1), dl) - 1)
        c_lo = jax.lax.shift_left(g, dl + 1) + o
        row_lo = c_lo * ch
        row_hi = row_lo + jax.lax.shift_left(jnp.int32(1), j_log)
        a = zref[pl.ds(row_lo, ch), :]
        b = zref[pl.ds(row_hi, ch), :]
        zref[pl.ds(row_lo, ch), :] = jnp.minimum(a, b)
        zref[pl.ds(row_hi, ch), :] = jnp.maximum(a, b)

    def chunk_tail(c, carry):
        row0 = c * ch
        a = zref[pl.ds(row0, ch), :]
        for j_log in range(log_ch - 1, -1, -1):
            a = _static_pass(a, 1 << j_log, nseg)
        zref[pl.ds(row0, ch), :] = a
        return carry

    def stage(k_log, carry):
        desc = (jax.lax.shift_right_logical(idx, k_log) & 1) == 1
        sgn = jnp.where(desc, jnp.float32(-1.0), jnp.float32(1.0))
        zref[...] = zref[...] * sgn
        n_lane = k_log - log_seg
        def lane_body(i, cc):
            def inner(c, c2):
                lane_pass_chunk(c, k_log - 1 - log_seg - i)
                return c2
            jax.lax.fori_loop(0, nch, inner, 0)
            return cc
        jax.lax.fori_loop(0, n_lane, lane_body, 0)
        n_big = jnp.maximum(jnp.minimum(k_log, log_seg) - log_ch, 0)
        def big_body(i, cc):
            def inner(t, c2):
                big_pass_pair(t, jnp.minimum(k_log, log_seg) - 1 - i)
                return c2
            jax.lax.fori_loop(0, nch // 2, inner, 0)
            return cc
        jax.lax.fori_loop(0, n_big, big_body, 0)
        jax.lax.fori_loop(0, nch, chunk_tail, 0)
        zref[...] = zref[...] * sgn
        return carry

    jax.lax.fori_loop(log_ch + 1, log_n + 1, stage, 0)


def _make_proj_body(seg, pch, d, l):
    def body(x_ref, y_ref, th_ref, a_ref):
        th = th_ref[...]
        norm = jnp.sqrt(jnp.sum(th * th, axis=0, keepdims=True))
        thn = th / (norm + 1e-12)

        def proj_t(v):
            p = jax.lax.dot_general(
                v, thn, (((1,), (0,)), ((), ())),
                precision=jax.lax.Precision.HIGHEST)
            return p.T.reshape(l, pch, seg)

        a_ref[0:l] = proj_t(x_ref[...])
        a_ref[l:2 * l] = proj_t(y_ref[...])

    return body


def _make_sort_body(seg, nseg, l):
    def body(ax_ref, ay_ref, out_ref, zref, xres):
        _sort_column(ax_ref[0], zref, seg, nseg)
        xres[...] = zref[...]
        _sort_column(ay_ref[0], zref, seg, nseg)
        out_ref[...] = jnp.sum(
            jnp.abs(xres[...] - zref[...])).reshape(1, 1, 1)

    return body


def _gswd(x, y, theta, seg, nseg, pch, d, l):
    n = seg * nseg
    a = pl.pallas_call(
        _make_proj_body(seg, pch, d, l),
        grid=(nseg // pch,),
        in_specs=[
            pl.BlockSpec((pch * seg, d), lambda i: (i, 0)),
            pl.BlockSpec((pch * seg, d), lambda i: (i, 0)),
            pl.BlockSpec((d, l), lambda i: (0, 0)),
        ],
        out_specs=pl.BlockSpec((2 * l, pch, seg), lambda i: (0, i, 0)),
        out_shape=jax.ShapeDtypeStruct((2 * l, nseg, seg), jnp.float32),
        compiler_params=pltpu.CompilerParams(
            dimension_semantics=("parallel",),
        ),
    )(x, y, theta)

    partials = pl.pallas_call(
        _make_sort_body(seg, nseg, l),
        grid=(l,),
        in_specs=[
            pl.BlockSpec((1, nseg, seg), lambda c: (c, 0, 0)),
            pl.BlockSpec((1, nseg, seg), lambda c: (c + l, 0, 0)),
        ],
        out_specs=pl.BlockSpec((1, 1, 1), lambda c: (c, 0, 0)),
        out_shape=jax.ShapeDtypeStruct((l, 1, 1), jnp.float32),
        scratch_shapes=[
            pltpu.VMEM((seg, nseg), jnp.float32),
            pltpu.VMEM((seg, nseg), jnp.float32),
        ],
        compiler_params=pltpu.CompilerParams(
            dimension_semantics=("parallel",),
        ),
    )(a, a)
    return jnp.sum(partials) / (n * l)


def kernel(x, y, theta):
    return _gswd(x, y, theta, SEG, NSEG, PCH, D, L)


# speedup vs baseline: 3.2702x; 1.1223x over previous
import jax
import jax.numpy as jnp
from jax.experimental import pallas as pl
from jax.experimental.pallas import tpu as pltpu

N = 131072
D = 64
L = 64
SEG = 1024
NSEG = 128
PCH = 8


def _log2(v):
    return v.bit_length() - 1


_CH = 256


def _static_pass(a, j, nseg):
    rows = a.shape[0]
    groups = rows // (2 * j)
    a4 = a.reshape(groups, 2, j, nseg)
    lower = a4[:, 0]
    upper = a4[:, 1]
    return jnp.stack(
        [jnp.minimum(lower, upper), jnp.maximum(lower, upper)],
        axis=1).reshape(rows, nseg)


def _sort_column(zt_in, zref, seg, nseg):
    log_seg = _log2(seg)
    log_n = _log2(seg * nseg)
    ch = min(_CH, seg)
    log_ch = _log2(ch)
    nch = seg // ch

    r_io = jax.lax.broadcasted_iota(jnp.int32, (seg, 1), 0)
    s_io = jax.lax.broadcasted_iota(jnp.int32, (1, nseg), 1)
    idx = s_io * seg + r_io

    zref[...] = zt_in.T

    rl_io = jax.lax.broadcasted_iota(jnp.int32, (ch, 1), 0)
    stage_sgn = {
        k: jnp.where((jax.lax.shift_right_logical(rl_io, k) & 1) == 1,
                     jnp.float32(-1.0), jnp.float32(1.0))
        for k in range(1, log_ch)
    }

    def chunk_sort(c, carry):
        row0 = c * ch
        a = zref[pl.ds(row0, ch), :]
        csgn = jnp.where(
            (jax.lax.shift_right_logical(s_io * seg + row0, log_ch) & 1) == 1,
            jnp.float32(-1.0), jnp.float32(1.0))
        a = a * csgn
        for k in range(1, log_ch + 1):
            if k < log_ch:
                a = a * stage_sgn[k]
            for j_log in range(k - 1, -1, -1):
                a = _static_pass(a, 1 << j_log, nseg)
            if k < log_ch:
                a = a * stage_sgn[k]
        a = a * csgn
        zref[pl.ds(row0, ch), :] = a
        return carry

    jax.lax.fori_loop(0, nch, chunk_sort, 0)

    def lane_pass_chunk(c, d_log):
        row0 = c * ch
        a = zref[pl.ds(row0, ch), :]
        d = jax.lax.shift_left(jnp.int32(1), d_log)
        up = pltpu.roll(a, nseg - d, 1)
        dn = pltpu.roll(a, d, 1)
        bitd = (jax.lax.shift_right_logical(s_io, d_log) & 1) == 1
        partner = jnp.where(bitd, dn, up)
        mn = jnp.minimum(a, partner)
        mx = jnp.maximum(a, partner)
        zref[pl.ds(row0, ch), :] = jnp.where(bitd, mx, mn)

    def big_pass_pair(t, j_log):
        dl = j_log - log_ch
        g = jax.lax.shift_right_logical(t, dl)
        o = t & (jax.lax.shift_left(jnp.int32(1), dl) - 1)
        c_lo = jax.lax.shift_left(g, dl + 1) + o
        row_lo = c_lo * ch
        row_hi = row_lo + jax.lax.shift_left(jnp.int32(1), j_log)
        a = zref[pl.ds(row_lo, ch), :]
        b = zref[pl.ds(row_hi, ch), :]
        zref[pl.ds(row_lo, ch), :] = jnp.minimum(a, b)
        zref[pl.ds(row_hi, ch), :] = jnp.maximum(a, b)

    def chunk_tail(c, carry):
        row0 = c * ch
        a = zref[pl.ds(row0, ch), :]
        for j_log in range(log_ch - 1, -1, -1):
            a = _static_pass(a, 1 << j_log, nseg)
        zref[pl.ds(row0, ch), :] = a
        return carry

    def stage(k_log, carry):
        desc = (jax.lax.shift_right_logical(idx, k_log) & 1) == 1
        sgn = jnp.where(desc, jnp.float32(-1.0), jnp.float32(1.0))
        zref[...] = zref[...] * sgn
        n_lane = k_log - log_seg
        def lane_body(i, cc):
            def inner(c, c2):
                lane_pass_chunk(c, k_log - 1 - log_seg - i)
                return c2
            jax.lax.fori_loop(0, nch, inner, 0)
            return cc
        jax.lax.fori_loop(0, n_lane, lane_body, 0)
        n_big = jnp.maximum(jnp.minimum(k_log, log_seg) - log_ch, 0)
        def big_body(i, cc):
            def inner(t, c2):
                big_pass_pair(t, jnp.minimum(k_log, log_seg) - 1 - i)
                return c2
            jax.lax.fori_loop(0, nch // 2, inner, 0)
            return cc
        jax.lax.fori_loop(0, n_big, big_body, 0)
        jax.lax.fori_loop(0, nch, chunk_tail, 0)
        zref[...] = zref[...] * sgn
        return carry

    jax.lax.fori_loop(log_ch + 1, log_n + 1, stage, 0)


def _make_proj_body(seg, pch, d, l):
    def body(x_ref, y_ref, th_ref, a_ref):
        th = th_ref[...]
        norm = jnp.sqrt(jnp.sum(th * th, axis=0, keepdims=True))
        thn = th / (norm + 1e-12)

        def proj_t(v):
            p = jax.lax.dot_general(
                v, thn, (((1,), (0,)), ((), ())),
                precision=jax.lax.Precision.HIGHEST)
            return p.T.reshape(l, pch, seg)

        a_ref[0:l] = proj_t(x_ref[...])
        a_ref[l:2 * l] = proj_t(y_ref[...])

    return body


def _make_sort_body(seg, nseg, l):
    def body(ax_ref, ay_ref, out_ref, zref, xres):
        _sort_column(ax_ref[0], zref, seg, nseg)
        xres[...] = zref[...]
        _sort_column(ay_ref[0], zref, seg, nseg)
        out_ref[...] = jnp.sum(
            jnp.abs(xres[...] - zref[...])).reshape(1, 1, 1)

    return body


def _gswd(x, y, theta, seg, nseg, pch, d, l):
    n = seg * nseg
    a = pl.pallas_call(
        _make_proj_body(seg, pch, d, l),
        grid=(nseg // pch,),
        in_specs=[
            pl.BlockSpec((pch * seg, d), lambda i: (i, 0)),
            pl.BlockSpec((pch * seg, d), lambda i: (i, 0)),
            pl.BlockSpec((d, l), lambda i: (0, 0)),
        ],
        out_specs=pl.BlockSpec((2 * l, pch, seg), lambda i: (0, i, 0)),
        out_shape=jax.ShapeDtypeStruct((2 * l, nseg, seg), jnp.float32),
        compiler_params=pltpu.CompilerParams(
            dimension_semantics=("parallel",),
        ),
    )(x, y, theta)

    partials = pl.pallas_call(
        _make_sort_body(seg, nseg, l),
        grid=(l,),
        in_specs=[
            pl.BlockSpec((1, nseg, seg), lambda c: (c, 0, 0)),
            pl.BlockSpec((1, nseg, seg), lambda c: (c + l, 0, 0)),
        ],
        out_specs=pl.BlockSpec((1, 1, 1), lambda c: (c, 0, 0)),
        out_shape=jax.ShapeDtypeStruct((l, 1, 1), jnp.float32),
        scratch_shapes=[
            pltpu.VMEM((seg, nseg), jnp.float32),
            pltpu.VMEM((seg, nseg), jnp.float32),
        ],
        compiler_params=pltpu.CompilerParams(
            dimension_semantics=("parallel",),
        ),
    )(a, a)
    return jnp.sum(partials) / (n * l)


def kernel(x, y, theta):
    return _gswd(x, y, theta, SEG, NSEG, PCH, D, L)


# speedup vs baseline: 3.2772x; 1.0021x over previous
import jax
import jax.numpy as jnp
from jax.experimental import pallas as pl
from jax.experimental.pallas import tpu as pltpu

N = 131072
D = 64
L = 64
SEG = 1024
NSEG = 128
PCH = 8


def _log2(v):
    return v.bit_length() - 1


_CH = 512


def _static_pass(a, j, nseg):
    rows = a.shape[0]
    groups = rows // (2 * j)
    a4 = a.reshape(groups, 2, j, nseg)
    lower = a4[:, 0]
    upper = a4[:, 1]
    return jnp.stack(
        [jnp.minimum(lower, upper), jnp.maximum(lower, upper)],
        axis=1).reshape(rows, nseg)


def _sort_column(zt_in, zref, seg, nseg):
    log_seg = _log2(seg)
    log_n = _log2(seg * nseg)
    ch = min(_CH, seg)
    log_ch = _log2(ch)
    nch = seg // ch

    r_io = jax.lax.broadcasted_iota(jnp.int32, (seg, 1), 0)
    s_io = jax.lax.broadcasted_iota(jnp.int32, (1, nseg), 1)
    idx = s_io * seg + r_io

    zref[...] = zt_in.T

    rl_io = jax.lax.broadcasted_iota(jnp.int32, (ch, 1), 0)
    stage_sgn = {
        k: jnp.where((jax.lax.shift_right_logical(rl_io, k) & 1) == 1,
                     jnp.float32(-1.0), jnp.float32(1.0))
        for k in range(1, log_ch)
    }

    def chunk_sort(c, carry):
        row0 = c * ch
        a = zref[pl.ds(row0, ch), :]
        csgn = jnp.where(
            (jax.lax.shift_right_logical(s_io * seg + row0, log_ch) & 1) == 1,
            jnp.float32(-1.0), jnp.float32(1.0))
        a = a * csgn
        for k in range(1, log_ch + 1):
            if k < log_ch:
                a = a * stage_sgn[k]
            for j_log in range(k - 1, -1, -1):
                a = _static_pass(a, 1 << j_log, nseg)
            if k < log_ch:
                a = a * stage_sgn[k]
        a = a * csgn
        zref[pl.ds(row0, ch), :] = a
        return carry

    jax.lax.fori_loop(0, nch, chunk_sort, 0)

    def lane_pass_chunk(c, d_log):
        row0 = c * ch
        a = zref[pl.ds(row0, ch), :]
        d = jax.lax.shift_left(jnp.int32(1), d_log)
        up = pltpu.roll(a, nseg - d, 1)
        dn = pltpu.roll(a, d, 1)
        bitd = (jax.lax.shift_right_logical(s_io, d_log) & 1) == 1
        partner = jnp.where(bitd, dn, up)
        mn = jnp.minimum(a, partner)
        mx = jnp.maximum(a, partner)
        zref[pl.ds(row0, ch), :] = jnp.where(bitd, mx, mn)

    def big_pass_pair(t, j_log):
        dl = j_log - log_ch
        g = jax.lax.shift_right_logical(t, dl)
        o = t & (jax.lax.shift_left(jnp.int32(1), dl) - 1)
        c_lo = jax.lax.shift_left(g, dl + 1) + o
        row_lo = c_lo * ch
        row_hi = row_lo + jax.lax.shift_left(jnp.int32(1), j_log)
        a = zref[pl.ds(row_lo, ch), :]
        b = zref[pl.ds(row_hi, ch), :]
        zref[pl.ds(row_lo, ch), :] = jnp.minimum(a, b)
        zref[pl.ds(row_hi, ch), :] = jnp.maximum(a, b)

    def chunk_tail(c, carry):
        row0 = c * ch
        a = zref[pl.ds(row0, ch), :]
        for j_log in range(log_ch - 1, -1, -1):
            a = _static_pass(a, 1 << j_log, nseg)
        zref[pl.ds(row0, ch), :] = a
        return carry

    def stage(k_log, carry):
        desc = (jax.lax.shift_right_logical(idx, k_log) & 1) == 1
        sgn = jnp.where(desc, jnp.float32(-1.0), jnp.float32(1.0))
        zref[...] = zref[...] * sgn
        n_lane = k_log - log_seg
        def lane_body(i, cc):
            def inner(c, c2):
                lane_pass_chunk(c, k_log - 1 - log_seg - i)
                return c2
            jax.lax.fori_loop(0, nch, inner, 0)
            return cc
        jax.lax.fori_loop(0, n_lane, lane_body, 0)
        n_big = jnp.maximum(jnp.minimum(k_log, log_seg) - log_ch, 0)
        def big_body(i, cc):
            def inner(t, c2):
                big_pass_pair(t, jnp.minimum(k_log, log_seg) - 1 - i)
                return c2
            jax.lax.fori_loop(0, nch // 2, inner, 0)
            return cc
        jax.lax.fori_loop(0, n_big, big_body, 0)
        jax.lax.fori_loop(0, nch, chunk_tail, 0)
        zref[...] = zref[...] * sgn
        return carry

    jax.lax.fori_loop(log_ch + 1, log_n + 1, stage, 0)


def _make_proj_body(seg, pch, d, l):
    def body(x_ref, y_ref, th_ref, a_ref):
        th = th_ref[...]
        norm = jnp.sqrt(jnp.sum(th * th, axis=0, keepdims=True))
        thn = th / (norm + 1e-12)

        def proj_t(v):
            p = jax.lax.dot_general(
                v, thn, (((1,), (0,)), ((), ())),
                precision=jax.lax.Precision.HIGHEST)
            return p.T.reshape(l, pch, seg)

        a_ref[0:l] = proj_t(x_ref[...])
        a_ref[l:2 * l] = proj_t(y_ref[...])

    return body


def _make_sort_body(seg, nseg, l):
    def body(ax_ref, ay_ref, out_ref, zref, xres):
        _sort_column(ax_ref[0], zref, seg, nseg)
        xres[...] = zref[...]
        _sort_column(ay_ref[0], zref, seg, nseg)
        out_ref[...] = jnp.sum(
            jnp.abs(xres[...] - zref[...])).reshape(1, 1, 1)

    return body


def _gswd(x, y, theta, seg, nseg, pch, d, l):
    n = seg * nseg
    a = pl.pallas_call(
        _make_proj_body(seg, pch, d, l),
        grid=(nseg // pch,),
        in_specs=[
            pl.BlockSpec((pch * seg, d), lambda i: (i, 0)),
            pl.BlockSpec((pch * seg, d), lambda i: (i, 0)),
            pl.BlockSpec((d, l), lambda i: (0, 0)),
        ],
        out_specs=pl.BlockSpec((2 * l, pch, seg), lambda i: (0, i, 0)),
        out_shape=jax.ShapeDtypeStruct((2 * l, nseg, seg), jnp.float32),
        compiler_params=pltpu.CompilerParams(
            dimension_semantics=("parallel",),
        ),
    )(x, y, theta)

    partials = pl.pallas_call(
        _make_sort_body(seg, nseg, l),
        grid=(l,),
        in_specs=[
            pl.BlockSpec((1, nseg, seg), lambda c: (c, 0, 0)),
            pl.BlockSpec((1, nseg, seg), lambda c: (c + l, 0, 0)),
        ],
        out_specs=pl.BlockSpec((1, 1, 1), lambda c: (c, 0, 0)),
        out_shape=jax.ShapeDtypeStruct((l, 1, 1), jnp.float32),
        scratch_shapes=[
            pltpu.VMEM((seg, nseg), jnp.float32),
            pltpu.VMEM((seg, nseg), jnp.float32),
        ],
        compiler_params=pltpu.CompilerParams(
            dimension_semantics=("parallel",),
        ),
    )(a, a)
    return jnp.sum(partials) / (n * l)


def kernel(x, y, theta):
    return _gswd(x, y, theta, SEG, NSEG, PCH, D, L)


# speedup vs baseline: 5.0693x; 1.5469x over previous
import jax
import jax.numpy as jnp
from jax.experimental import pallas as pl
from jax.experimental.pallas import tpu as pltpu

N = 131072
D = 64
L = 64
SEG = 1024
NSEG = 128
PCH = 8


def _log2(v):
    return v.bit_length() - 1


_CH = 512


def _static_pass(a, j_log, rl_io):
    rows = a.shape[0]
    j = 1 << j_log
    up = pltpu.roll(a, rows - j, 0)
    dn = pltpu.roll(a, j, 0)
    bitj = (jax.lax.shift_right_logical(rl_io, j_log) & 1) == 1
    partner = jnp.where(bitj, dn, up)
    mn = jnp.minimum(a, partner)
    mx = jnp.maximum(a, partner)
    return jnp.where(bitj, mx, mn)


def _sort_column(zt_in, zref, seg, nseg):
    log_seg = _log2(seg)
    log_n = _log2(seg * nseg)
    ch = min(_CH, seg)
    log_ch = _log2(ch)
    nch = seg // ch

    r_io = jax.lax.broadcasted_iota(jnp.int32, (seg, 1), 0)
    s_io = jax.lax.broadcasted_iota(jnp.int32, (1, nseg), 1)
    idx = s_io * seg + r_io

    zref[...] = zt_in.T

    rl_io = jax.lax.broadcasted_iota(jnp.int32, (ch, 1), 0)
    stage_sgn = {
        k: jnp.where((jax.lax.shift_right_logical(rl_io, k) & 1) == 1,
                     jnp.float32(-1.0), jnp.float32(1.0))
        for k in range(1, log_ch)
    }

    def chunk_sort(c, carry):
        row0 = c * ch
        a = zref[pl.ds(row0, ch), :]
        csgn = jnp.where(
            (jax.lax.shift_right_logical(s_io * seg + row0, log_ch) & 1) == 1,
            jnp.float32(-1.0), jnp.float32(1.0))
        a = a * csgn
        for k in range(1, log_ch + 1):
            if k < log_ch:
                a = a * stage_sgn[k]
            for j_log in range(k - 1, -1, -1):
                a = _static_pass(a, j_log, rl_io)
            if k < log_ch:
                a = a * stage_sgn[k]
        a = a * csgn
        zref[pl.ds(row0, ch), :] = a
        return carry

    jax.lax.fori_loop(0, nch, chunk_sort, 0)

    def lane_pass_chunk(c, d_log):
        row0 = c * ch
        a = zref[pl.ds(row0, ch), :]
        d = jax.lax.shift_left(jnp.int32(1), d_log)
        up = pltpu.roll(a, nseg - d, 1)
        dn = pltpu.roll(a, d, 1)
        bitd = (jax.lax.shift_right_logical(s_io, d_log) & 1) == 1
        partner = jnp.where(bitd, dn, up)
        mn = jnp.minimum(a, partner)
        mx = jnp.maximum(a, partner)
        zref[pl.ds(row0, ch), :] = jnp.where(bitd, mx, mn)

    def big_pass_pair(t, j_log):
        dl = j_log - log_ch
        g = jax.lax.shift_right_logical(t, dl)
        o = t & (jax.lax.shift_left(jnp.int32(1), dl) - 1)
        c_lo = jax.lax.shift_left(g, dl + 1) + o
        row_lo = c_lo * ch
        row_hi = row_lo + jax.lax.shift_left(jnp.int32(1), j_log)
        a = zref[pl.ds(row_lo, ch), :]
        b = zref[pl.ds(row_hi, ch), :]
        zref[pl.ds(row_lo, ch), :] = jnp.minimum(a, b)
        zref[pl.ds(row_hi, ch), :] = jnp.maximum(a, b)

    def chunk_tail(c, carry):
        row0 = c * ch
        a = zref[pl.ds(row0, ch), :]
        for j_log in range(log_ch - 1, -1, -1):
            a = _static_pass(a, j_log, rl_io)
        zref[pl.ds(row0, ch), :] = a
        return carry

    def stage(k_log, carry):
        desc = (jax.lax.shift_right_logical(idx, k_log) & 1) == 1
        sgn = jnp.where(desc, jnp.float32(-1.0), jnp.float32(1.0))
        zref[...] = zref[...] * sgn
        n_lane = k_log - log_seg
        def lane_body(i, cc):
            def inner(c, c2):
                lane_pass_chunk(c, k_log - 1 - log_seg - i)
                return c2
            jax.lax.fori_loop(0, nch, inner, 0)
            return cc
        jax.lax.fori_loop(0, n_lane, lane_body, 0)
        n_big = jnp.maximum(jnp.minimum(k_log, log_seg) - log_ch, 0)
        def big_body(i, cc):
            def inner(t, c2):
                big_pass_pair(t, jnp.minimum(k_log, log_seg) - 1 - i)
                return c2
            jax.lax.fori_loop(0, nch // 2, inner, 0)
            return cc
        jax.lax.fori_loop(0, n_big, big_body, 0)
        jax.lax.fori_loop(0, nch, chunk_tail, 0)
        zref[...] = zref[...] * sgn
        return carry

    jax.lax.fori_loop(log_ch + 1, log_n + 1, stage, 0)


def _make_proj_body(seg, pch, d, l):
    def body(x_ref, y_ref, th_ref, a_ref):
        th = th_ref[...]
        norm = jnp.sqrt(jnp.sum(th * th, axis=0, keepdims=True))
        thn = th / (norm + 1e-12)

        def proj_t(v):
            p = jax.lax.dot_general(
                v, thn, (((1,), (0,)), ((), ())),
                precision=jax.lax.Precision.HIGHEST)
            return p.T.reshape(l, pch, seg)

        a_ref[0:l] = proj_t(x_ref[...])
        a_ref[l:2 * l] = proj_t(y_ref[...])

    return body


def _make_sort_body(seg, nseg, l):
    def body(ax_ref, ay_ref, out_ref, zref, xres):
        _sort_column(ax_ref[0], zref, seg, nseg)
        xres[...] = zref[...]
        _sort_column(ay_ref[0], zref, seg, nseg)
        out_ref[...] = jnp.sum(
            jnp.abs(xres[...] - zref[...])).reshape(1, 1, 1)

    return body


def _gswd(x, y, theta, seg, nseg, pch, d, l):
    n = seg * nseg
    a = pl.pallas_call(
        _make_proj_body(seg, pch, d, l),
        grid=(nseg // pch,),
        in_specs=[
            pl.BlockSpec((pch * seg, d), lambda i: (i, 0)),
            pl.BlockSpec((pch * seg, d), lambda i: (i, 0)),
            pl.BlockSpec((d, l), lambda i: (0, 0)),
        ],
        out_specs=pl.BlockSpec((2 * l, pch, seg), lambda i: (0, i, 0)),
        out_shape=jax.ShapeDtypeStruct((2 * l, nseg, seg), jnp.float32),
        compiler_params=pltpu.CompilerParams(
            dimension_semantics=("parallel",),
        ),
    )(x, y, theta)

    partials = pl.pallas_call(
        _make_sort_body(seg, nseg, l),
        grid=(l,),
        in_specs=[
            pl.BlockSpec((1, nseg, seg), lambda c: (c, 0, 0)),
            pl.BlockSpec((1, nseg, seg), lambda c: (c + l, 0, 0)),
        ],
        out_specs=pl.BlockSpec((1, 1, 1), lambda c: (c, 0, 0)),
        out_shape=jax.ShapeDtypeStruct((l, 1, 1), jnp.float32),
        scratch_shapes=[
            pltpu.VMEM((seg, nseg), jnp.float32),
            pltpu.VMEM((seg, nseg), jnp.float32),
        ],
        compiler_params=pltpu.CompilerParams(
            dimension_semantics=("parallel",),
        ),
    )(a, a)
    return jnp.sum(partials) / (n * l)


def kernel(x, y, theta):
    return _gswd(x, y, theta, SEG, NSEG, PCH, D, L)


# speedup vs baseline: 5.2639x; 1.0384x over previous
import jax
import jax.numpy as jnp
from jax.experimental import pallas as pl
from jax.experimental.pallas import tpu as pltpu

N = 131072
D = 64
L = 64
SEG = 1024
NSEG = 128
PCH = 8


def _log2(v):
    return v.bit_length() - 1


_CH = 512


def _fxor(v, m):
    return jax.lax.bitcast_convert_type(
        jax.lax.bitcast_convert_type(v, jnp.int32) ^ m, jnp.float32)


def _static_pass(a, j_log, rl_io):
    rows = a.shape[0]
    j = 1 << j_log
    u = pltpu.roll(a, rows - j, 0)
    mn = jnp.minimum(a, u)
    mx = jnp.maximum(a, u)
    mxr = pltpu.roll(mx, j, 0)
    bitj = (jax.lax.shift_right_logical(rl_io, j_log) & 1) == 1
    return jnp.where(bitj, mxr, mn)


def _sort_column(zt_in, zref, seg, nseg):
    log_seg = _log2(seg)
    log_n = _log2(seg * nseg)
    ch = min(_CH, seg)
    log_ch = _log2(ch)
    nch = seg // ch

    r_io = jax.lax.broadcasted_iota(jnp.int32, (seg, 1), 0)
    s_io = jax.lax.broadcasted_iota(jnp.int32, (1, nseg), 1)
    idx = s_io * seg + r_io

    zref[...] = zt_in.T

    rl_io = jax.lax.broadcasted_iota(jnp.int32, (ch, 1), 0)
    stage_msk = {k: jax.lax.shift_left(rl_io, 31 - k)
                 for k in range(1, log_ch)}

    def chunk_sort(c, carry):
        row0 = c * ch
        a = zref[pl.ds(row0, ch), :]
        cmsk = jax.lax.shift_left(s_io * seg + row0, 31 - log_ch)
        a = _fxor(a, cmsk)
        for k in range(1, log_ch + 1):
            if k < log_ch:
                a = _fxor(a, stage_msk[k])
            for j_log in range(k - 1, -1, -1):
                a = _static_pass(a, j_log, rl_io)
            if k < log_ch:
                a = _fxor(a, stage_msk[k])
        a = _fxor(a, cmsk)
        zref[pl.ds(row0, ch), :] = a
        return carry

    jax.lax.fori_loop(0, nch, chunk_sort, 0)

    def lane_pass_chunk(c, d_log):
        row0 = c * ch
        a = zref[pl.ds(row0, ch), :]
        d = jax.lax.shift_left(jnp.int32(1), d_log)
        u = pltpu.roll(a, nseg - d, 1)
        mn = jnp.minimum(a, u)
        mx = jnp.maximum(a, u)
        mxr = pltpu.roll(mx, d, 1)
        bitd = (jax.lax.shift_right_logical(s_io, d_log) & 1) == 1
        zref[pl.ds(row0, ch), :] = jnp.where(bitd, mxr, mn)

    def big_pass_pair(t, j_log):
        dl = j_log - log_ch
        g = jax.lax.shift_right_logical(t, dl)
        o = t & (jax.lax.shift_left(jnp.int32(1), dl) - 1)
        c_lo = jax.lax.shift_left(g, dl + 1) + o
        row_lo = c_lo * ch
        row_hi = row_lo + jax.lax.shift_left(jnp.int32(1), j_log)
        a = zref[pl.ds(row_lo, ch), :]
        b = zref[pl.ds(row_hi, ch), :]
        zref[pl.ds(row_lo, ch), :] = jnp.minimum(a, b)
        zref[pl.ds(row_hi, ch), :] = jnp.maximum(a, b)

    def chunk_tail(c, carry):
        row0 = c * ch
        a = zref[pl.ds(row0, ch), :]
        for j_log in range(log_ch - 1, -1, -1):
            a = _static_pass(a, j_log, rl_io)
        zref[pl.ds(row0, ch), :] = a
        return carry

    zref[...] = _fxor(zref[...], jax.lax.shift_left(idx, 31 - (log_ch + 1)))

    def stage(k_log, carry):
        n_lane = k_log - log_seg
        def lane_body(i, cc):
            def inner(c, c2):
                lane_pass_chunk(c, k_log - 1 - log_seg - i)
                return c2
            jax.lax.fori_loop(0, nch, inner, 0)
            return cc
        jax.lax.fori_loop(0, n_lane, lane_body, 0)
        n_big = jnp.maximum(jnp.minimum(k_log, log_seg) - log_ch, 0)
        def big_body(i, cc):
            def inner(t, c2):
                big_pass_pair(t, jnp.minimum(k_log, log_seg) - 1 - i)
                return c2
            jax.lax.fori_loop(0, nch // 2, inner, 0)
            return cc
        jax.lax.fori_loop(0, n_big, big_body, 0)
        jax.lax.fori_loop(0, nch, chunk_tail, 0)
        bmask = (jax.lax.shift_left(idx, 31 - k_log)
                 ^ jax.lax.shift_left(idx, 30 - k_log))
        zref[...] = _fxor(zref[...], bmask)
        return carry

    jax.lax.fori_loop(log_ch + 1, log_n + 1, stage, 0)


def _make_proj_body(seg, pch, d, l):
    def body(x_ref, y_ref, th_ref, a_ref):
        th = th_ref[...]
        norm = jnp.sqrt(jnp.sum(th * th, axis=0, keepdims=True))
        thn = th / (norm + 1e-12)

        def proj_t(v):
            p = jax.lax.dot_general(
                v, thn, (((1,), (0,)), ((), ())),
                precision=jax.lax.Precision.HIGHEST)
            return p.T.reshape(l, pch, seg)

        a_ref[0:l] = proj_t(x_ref[...])
        a_ref[l:2 * l] = proj_t(y_ref[...])

    return body


def _make_sort_body(seg, nseg, l):
    def body(ax_ref, ay_ref, out_ref, zref, xres):
        _sort_column(ax_ref[0], zref, seg, nseg)
        xres[...] = zref[...]
        _sort_column(ay_ref[0], zref, seg, nseg)
        out_ref[...] = jnp.sum(
            jnp.abs(xres[...] - zref[...])).reshape(1, 1, 1)

    return body


def _gswd(x, y, theta, seg, nseg, pch, d, l):
    n = seg * nseg
    a = pl.pallas_call(
        _make_proj_body(seg, pch, d, l),
        grid=(nseg // pch,),
        in_specs=[
            pl.BlockSpec((pch * seg, d), lambda i: (i, 0)),
            pl.BlockSpec((pch * seg, d), lambda i: (i, 0)),
            pl.BlockSpec((d, l), lambda i: (0, 0)),
        ],
        out_specs=pl.BlockSpec((2 * l, pch, seg), lambda i: (0, i, 0)),
        out_shape=jax.ShapeDtypeStruct((2 * l, nseg, seg), jnp.float32),
        compiler_params=pltpu.CompilerParams(
            dimension_semantics=("parallel",),
        ),
    )(x, y, theta)

    partials = pl.pallas_call(
        _make_sort_body(seg, nseg, l),
        grid=(l,),
        in_specs=[
            pl.BlockSpec((1, nseg, seg), lambda c: (c, 0, 0)),
            pl.BlockSpec((1, nseg, seg), lambda c: (c + l, 0, 0)),
        ],
        out_specs=pl.BlockSpec((1, 1, 1), lambda c: (c, 0, 0)),
        out_shape=jax.ShapeDtypeStruct((l, 1, 1), jnp.float32),
        scratch_shapes=[
            pltpu.VMEM((seg, nseg), jnp.float32),
            pltpu.VMEM((seg, nseg), jnp.float32),
        ],
        compiler_params=pltpu.CompilerParams(
            dimension_semantics=("parallel",),
        ),
    )(a, a)
    return jnp.sum(partials) / (n * l)


def kernel(x, y, theta):
    return _gswd(x, y, theta, SEG, NSEG, PCH, D, L)
